# baseline (speedup 1.0000x reference)
import functools

import jax
import jax.numpy as jnp
from jax import lax
from jax.experimental import pallas as pl
from jax.experimental.pallas import tpu as pltpu
from jax.experimental.pallas import tpu_sc as plsc

NUM_SLOTS = 1024
NUM_ROWS = 32768

NC = 2
NS = 16
NW = NC * NS
RPW = NUM_SLOTS // NW

COPY_BLOCK = 512


N_COPY_CHUNKS = 16
COPY_CHUNK = NUM_ROWS // N_COPY_CHUNKS


def _tc_copy_body(src_ref, dst_ref, sems):
    for i in range(N_COPY_CHUNKS):
        pltpu.make_async_copy(
            src_ref.at[pl.ds(i * COPY_CHUNK, COPY_CHUNK)],
            dst_ref.at[pl.ds(i * COPY_CHUNK, COPY_CHUNK)],
            sems.at[i],
        ).start()
    for i in range(N_COPY_CHUNKS):
        pltpu.make_async_copy(
            src_ref.at[pl.ds(i * COPY_CHUNK, COPY_CHUNK)],
            dst_ref.at[pl.ds(i * COPY_CHUNK, COPY_CHUNK)],
            sems.at[i],
        ).wait()


def _tc_copy(cache):
    return pl.pallas_call(
        _tc_copy_body,
        in_specs=[pl.BlockSpec(memory_space=pl.ANY)],
        out_specs=pl.BlockSpec(memory_space=pl.ANY),
        scratch_shapes=[pltpu.SemaphoreType.DMA((N_COPY_CHUNKS,))],
        out_shape=jax.ShapeDtypeStruct((NUM_ROWS, 8, 128), cache.dtype),
    )(cache)


_sc_mesh = plsc.VectorSubcoreMesh(core_axis_name="c", subcore_axis_name="s")


@functools.partial(
    pl.kernel,
    mesh=_sc_mesh,
    scratch_types=[
        pltpu.VMEM((RPW,), jnp.int32),
        pltpu.VMEM((RPW, 8, 128), jnp.float32),
        pltpu.SemaphoreType.DMA,
    ],
)
def _sc_scatter(inp_hbm, sm_hbm, out_ref, idx_v, rows_v, sem):
    wid = lax.axis_index("s") * NC + lax.axis_index("c")
    base = wid * RPW
    pltpu.sync_copy(sm_hbm.at[pl.ds(base, RPW)], idx_v)
    pltpu.sync_copy(inp_hbm.at[pl.ds(base, RPW)], rows_v)
    pltpu.async_copy(rows_v, out_ref.at[idx_v], sem).wait()


def kernel(input, cache, slot_mapping):
    out = _tc_copy(cache)
    ref = jax.new_ref(out)
    _sc_scatter(input, slot_mapping.astype(jnp.int32), ref)
    return ref[...]


# speedup vs baseline: 33.8901x; 33.8901x over previous
import functools

import jax
import jax.numpy as jnp
from jax import lax
from jax.experimental import pallas as pl
from jax.experimental.pallas import tpu as pltpu
from jax.experimental.pallas import tpu_sc as plsc

NUM_SLOTS = 1024
NUM_ROWS = 32768

NC = 2
NS = 16
NW = NC * NS
RPW = NUM_SLOTS // NW

CHUNK = 32
CPW = NUM_ROWS // (NW * CHUNK)

_sc_mesh = plsc.VectorSubcoreMesh(core_axis_name="c", subcore_axis_name="s")


@functools.partial(
    pl.kernel,
    out_type=jax.ShapeDtypeStruct((NUM_ROWS, 8, 128), jnp.float32),
    mesh=_sc_mesh,
    compiler_params=pltpu.CompilerParams(needs_layout_passes=False),
    scratch_types=[
        pltpu.VMEM((2, CHUNK, 8, 128), jnp.float32),
        pltpu.SemaphoreType.DMA((2,)),
        pltpu.SemaphoreType.DMA((2,)),
    ],
)
def _sc_copy(cache, out, bufs, load_sems, store_sems):
    w = lax.axis_index("s") * NC + lax.axis_index("c")

    def chunk_rows(ci):
        return pl.ds((w + ci * NW) * CHUNK, CHUNK)

    def load(ci):
        return pltpu.make_async_copy(
            cache.at[chunk_rows(ci)], bufs.at[ci % 2], load_sems.at[ci % 2]
        )

    def store(ci):
        return pltpu.make_async_copy(
            bufs.at[ci % 2], out.at[chunk_rows(ci)], store_sems.at[ci % 2]
        )

    load(0).start()
    for ci in range(CPW):
        if ci >= 1:
            store(ci - 1).wait()
        if ci + 1 < CPW:
            load(ci + 1).start()
        load(ci).wait()
        store(ci).start()
    store(CPW - 1).wait()


@functools.partial(
    pl.kernel,
    mesh=_sc_mesh,
    scratch_types=[
        pltpu.VMEM((RPW,), jnp.int32),
        pltpu.VMEM((RPW, 8, 128), jnp.float32),
        pltpu.SemaphoreType.DMA,
    ],
)
def _sc_scatter(inp_hbm, sm_hbm, out_ref, idx_v, rows_v, sem):
    wid = lax.axis_index("s") * NC + lax.axis_index("c")
    base = wid * RPW
    pltpu.sync_copy(sm_hbm.at[pl.ds(base, RPW)], idx_v)
    pltpu.sync_copy(inp_hbm.at[pl.ds(base, RPW)], rows_v)
    pltpu.async_copy(rows_v, out_ref.at[idx_v], sem).wait()


def kernel(input, cache, slot_mapping):
    out = _sc_copy(cache)
    ref = jax.new_ref(out)
    _sc_scatter(input, slot_mapping.astype(jnp.int32), ref)
    return ref[...]


# speedup vs baseline: 39.1142x; 1.1541x over previous
import functools

import jax
import jax.numpy as jnp
from jax import lax
from jax.experimental import pallas as pl
from jax.experimental.pallas import tpu as pltpu
from jax.experimental.pallas import tpu_sc as plsc

NUM_SLOTS = 1024
NUM_ROWS = 32768

NC = 2
NS = 16
NW = NC * NS
RPW = NUM_SLOTS // NW

COPY_BLOCK = 1024


def _tc_copy_body(src_ref, dst_ref):
    dst_ref[...] = src_ref[...]


def _tc_copy(cache):
    return pl.pallas_call(
        _tc_copy_body,
        grid=(NUM_ROWS // COPY_BLOCK,),
        in_specs=[pl.BlockSpec((COPY_BLOCK, 8, 128), lambda i: (i, 0, 0))],
        out_specs=pl.BlockSpec((COPY_BLOCK, 8, 128), lambda i: (i, 0, 0)),
        out_shape=jax.ShapeDtypeStruct((NUM_ROWS, 8, 128), cache.dtype),
    )(cache)


_sc_mesh = plsc.VectorSubcoreMesh(core_axis_name="c", subcore_axis_name="s")


@functools.partial(
    pl.kernel,
    mesh=_sc_mesh,
    scratch_types=[
        pltpu.VMEM((RPW,), jnp.int32),
        pltpu.VMEM((RPW, 8, 128), jnp.float32),
        pltpu.SemaphoreType.DMA,
    ],
)
def _sc_scatter(inp_hbm, sm_hbm, out_ref, idx_v, rows_v, sem):
    wid = lax.axis_index("s") * NC + lax.axis_index("c")
    base = wid * RPW
    pltpu.sync_copy(sm_hbm.at[pl.ds(base, RPW)], idx_v)
    pltpu.sync_copy(inp_hbm.at[pl.ds(base, RPW)], rows_v)
    pltpu.async_copy(rows_v, out_ref.at[idx_v], sem).wait()


def kernel(input, cache, slot_mapping):
    out = _tc_copy(cache)
    ref = jax.new_ref(out)
    _sc_scatter(input, slot_mapping.astype(jnp.int32), ref)
    return ref[...]


# speedup vs baseline: 39.7075x; 1.0152x over previous
import functools

import jax
import jax.numpy as jnp
from jax import lax
from jax.experimental import pallas as pl
from jax.experimental.pallas import tpu as pltpu
from jax.experimental.pallas import tpu_sc as plsc

NUM_SLOTS = 1024
NUM_ROWS = 32768

NC = 2
NS = 16
NW = NC * NS
RPW = NUM_SLOTS // NW

COPY_BLOCK = 2048


def _tc_copy_body(src_ref, dst_ref):
    dst_ref[...] = src_ref[...]


def _tc_copy(cache):
    return pl.pallas_call(
        _tc_copy_body,
        grid=(NUM_ROWS // COPY_BLOCK,),
        in_specs=[pl.BlockSpec((COPY_BLOCK, 8, 128), lambda i: (i, 0, 0))],
        out_specs=pl.BlockSpec((COPY_BLOCK, 8, 128), lambda i: (i, 0, 0)),
        out_shape=jax.ShapeDtypeStruct((NUM_ROWS, 8, 128), cache.dtype),
    )(cache)


_sc_mesh = plsc.VectorSubcoreMesh(core_axis_name="c", subcore_axis_name="s")


@functools.partial(
    pl.kernel,
    mesh=_sc_mesh,
    scratch_types=[
        pltpu.VMEM((RPW,), jnp.int32),
        pltpu.VMEM((RPW, 8, 128), jnp.float32),
        pltpu.SemaphoreType.DMA,
    ],
)
def _sc_scatter(inp_hbm, sm_hbm, out_ref, idx_v, rows_v, sem):
    wid = lax.axis_index("s") * NC + lax.axis_index("c")
    base = wid * RPW
    pltpu.sync_copy(sm_hbm.at[pl.ds(base, RPW)], idx_v)
    pltpu.sync_copy(inp_hbm.at[pl.ds(base, RPW)], rows_v)
    pltpu.async_copy(rows_v, out_ref.at[idx_v], sem).wait()


def kernel(input, cache, slot_mapping):
    out = _tc_copy(cache)
    ref = jax.new_ref(out)
    _sc_scatter(input, slot_mapping.astype(jnp.int32), ref)
    return ref[...]


# speedup vs baseline: 39.8167x; 1.0027x over previous
import functools

import jax
import jax.numpy as jnp
from jax import lax
from jax.experimental import pallas as pl
from jax.experimental.pallas import tpu as pltpu
from jax.experimental.pallas import tpu_sc as plsc

NUM_SLOTS = 1024
NUM_ROWS = 32768

NC = 2
NS = 16
NW = NC * NS
RPW = NUM_SLOTS // NW

COPY_BLOCK = 2048


def _tc_copy_body(src_ref, dst_ref):
    dst_ref[...] = src_ref[...]


def _tc_copy(cache):
    return pl.pallas_call(
        _tc_copy_body,
        grid=(NUM_ROWS // COPY_BLOCK,),
        in_specs=[pl.BlockSpec((COPY_BLOCK, 8, 128), lambda i: (i, 0, 0))],
        out_specs=pl.BlockSpec((COPY_BLOCK, 8, 128), lambda i: (i, 0, 0)),
        out_shape=jax.ShapeDtypeStruct((NUM_ROWS, 8, 128), cache.dtype),
    )(cache)


_sc_mesh = plsc.VectorSubcoreMesh(core_axis_name="c", subcore_axis_name="s")


@functools.partial(
    pl.kernel,
    mesh=_sc_mesh,
    scratch_types=[
        pltpu.VMEM((RPW,), jnp.int32),
        pltpu.VMEM((RPW, 8, 128), jnp.float32),
        pltpu.SemaphoreType.DMA,
        pltpu.SemaphoreType.DMA,
        pltpu.SemaphoreType.DMA,
    ],
)
def _sc_scatter(inp_hbm, sm_hbm, out_ref, idx_v, rows_v, sem, sem_i, sem_r):
    wid = lax.axis_index("s") * NC + lax.axis_index("c")
    base = wid * RPW
    ci = pltpu.make_async_copy(sm_hbm.at[pl.ds(base, RPW)], idx_v, sem_i)
    cr = pltpu.make_async_copy(inp_hbm.at[pl.ds(base, RPW)], rows_v, sem_r)
    ci.start()
    cr.start()
    ci.wait()
    cr.wait()
    pltpu.async_copy(rows_v, out_ref.at[idx_v], sem).wait()


def kernel(input, cache, slot_mapping):
    out = _tc_copy(cache)
    ref = jax.new_ref(out)
    _sc_scatter(input, slot_mapping.astype(jnp.int32), ref)
    return ref[...]
